# baseline (speedup 1.0000x reference)
import functools

import jax
import jax.numpy as jnp
from jax import lax
from jax.experimental import pallas as pl
from jax.experimental.pallas import tpu as pltpu
from jax.experimental.pallas import tpu_sc as plsc

_CHUNK = 128


@functools.lru_cache(maxsize=None)
def _make_gather(vocab, dim, batch):
    info = plsc.get_sparse_core_info()
    num_cores, num_subcores = info.num_cores, info.num_subcores
    num_workers = num_cores * num_subcores
    b_per_w = batch // num_workers
    assert b_per_w * num_workers == batch
    n_chunks = b_per_w // _CHUNK
    assert n_chunks * _CHUNK == b_per_w

    mesh = plsc.VectorSubcoreMesh(core_axis_name="c", subcore_axis_name="s")

    @functools.partial(
        pl.kernel,
        mesh=mesh,
        compiler_params=pltpu.CompilerParams(use_tc_tiling_on_sc=False),
        out_type=jax.ShapeDtypeStruct((num_workers, n_chunks, _CHUNK, dim),
                                      jnp.float32),
        scratch_types=[
            pltpu.VMEM((n_chunks, _CHUNK), jnp.int32),
            pltpu.VMEM((n_chunks, _CHUNK, dim), jnp.float32),
            pltpu.SemaphoreType.DMA,
        ],
    )
    def gather_kernel(table_hbm, idx_hbm, out_hbm, idx_v, rows_v, sem):
        wid = lax.axis_index("s") * num_cores + lax.axis_index("c")
        pltpu.sync_copy(idx_hbm.at[wid], idx_v)
        copies = [
            pltpu.async_copy(table_hbm.at[idx_v.at[j]], rows_v.at[j], sem)
            for j in range(n_chunks)
        ]
        for c in copies:
            c.wait()
        pltpu.sync_copy(rows_v, out_hbm.at[wid])

    return gather_kernel


def kernel(e, weight):
    batch = e.shape[0]
    vocab, dim = weight.shape
    info = plsc.get_sparse_core_info()
    num_workers = info.num_cores * info.num_subcores
    n_chunks = batch // (num_workers * _CHUNK)
    idx = e.astype(jnp.int32).reshape(num_workers, n_chunks, _CHUNK)
    out = _make_gather(vocab, dim, batch)(weight, idx)
    return out.reshape(batch, 1, dim)


# speedup vs baseline: 1.0058x; 1.0058x over previous
import functools

import jax
import jax.numpy as jnp
from jax import lax
from jax.experimental import pallas as pl
from jax.experimental.pallas import tpu as pltpu
from jax.experimental.pallas import tpu_sc as plsc

_CHUNK = 128
_LANES = 16
_BLOCK = 8


@functools.lru_cache(maxsize=None)
def _make_gather(vocab, dim, batch):
    info = plsc.get_sparse_core_info()
    num_cores, num_subcores = info.num_cores, info.num_subcores
    num_workers = num_cores * num_subcores
    b_per_w = batch // num_workers
    assert b_per_w * num_workers == batch
    n_chunks = b_per_w // _CHUNK
    assert n_chunks * _CHUNK == b_per_w
    groups_per_chunk = _CHUNK // _LANES

    mesh = plsc.VectorSubcoreMesh(core_axis_name="c", subcore_axis_name="s")

    @functools.partial(
        pl.kernel,
        mesh=mesh,
        compiler_params=pltpu.CompilerParams(needs_layout_passes=False),
        out_type=jax.ShapeDtypeStruct((dim, batch), jnp.float32),
        scratch_types=[
            pltpu.VMEM((b_per_w,), jnp.int32),
            pltpu.VMEM((n_chunks, _CHUNK), jnp.int32),
            pltpu.VMEM((b_per_w, _BLOCK * dim), jnp.float32),
            pltpu.VMEM((dim, b_per_w), jnp.float32),
            [pltpu.SemaphoreType.DMA] * 1,
            [pltpu.SemaphoreType.DMA] * 8,
        ],
    )
    def gather_kernel(table_hbm, idx_hbm, out_hbm, idx_v, blk_v, rows_v,
                      out_v, idx_sems, sems):
        wid = lax.axis_index("s") * num_cores + lax.axis_index("c")
        iota = lax.iota(jnp.int32, _LANES)
        pltpu.async_copy(idx_hbm.at[pl.ds(wid * b_per_w, b_per_w)], idx_v,
                         idx_sems[0]).wait()
        shift = _BLOCK.bit_length() - 1
        copies = []
        for c in range(n_chunks):
            for g in range(groups_per_chunk):
                s = idx_v[pl.ds(c * _CHUNK + g * _LANES, _LANES)]
                blk_v[c, pl.ds(g * _LANES, _LANES)] = s >> shift
            copies.append(pltpu.async_copy(
                table_hbm.at[blk_v.at[c]],
                rows_v.at[pl.ds(c * _CHUNK, _CHUNK)], sems[c]))
        for c in range(n_chunks):
            copies[c].wait()
            for g in range(groups_per_chunk):
                base = c * _CHUNK + g * _LANES
                s = idx_v[pl.ds(base, _LANES)]
                col0 = (s & (_BLOCK - 1)) * dim
                row_idx = base + iota
                for l in range(dim):
                    v = plsc.load_gather(rows_v, [row_idx, col0 + l])
                    plsc.store_scatter(
                        out_v, [jnp.full((_LANES,), l, jnp.int32),
                                base + iota], v)
        pltpu.sync_copy(
            out_v,
            out_hbm.at[:, pl.ds(pl.multiple_of(wid * b_per_w, _CHUNK),
                                b_per_w)])

    return gather_kernel


def kernel(e, weight):
    batch = e.shape[0]
    vocab, dim = weight.shape
    table = weight.reshape(vocab // _BLOCK, _BLOCK * dim)
    idx = e.astype(jnp.int32)
    out_t = _make_gather(vocab, dim, batch)(table, idx)
    return out_t.T.reshape(batch, 1, dim)
